# Initial kernel scaffold; baseline (speedup 1.0000x reference)
import jax
import jax.numpy as jnp
from jax.experimental import pallas as pl


def kernel(features, memory_features, centers, points):
    raise NotImplementedError("write your pallas kernel here")


# speedup vs baseline: 22.7651x; 22.7651x over previous
import functools

import jax
import jax.numpy as jnp
from jax.experimental import pallas as pl

Q = 4096
K = 16384
D = 256
P = 16384
KNN_K = 10

_QB = 512
_KB = 2048
_PB = 256
_W = 128

_BIG = 3e38


def _stage1_body(f_ref, m_ref, o_ref):
    j = pl.program_id(1)
    f = f_ref[...]
    m = m_ref[...]
    ones = jnp.ones((1, D), jnp.float32)
    m2 = jax.lax.dot_general(
        ones, m * m, (((1,), (1,)), ((), ())),
        preferred_element_type=jnp.float32)
    prod = jax.lax.dot_general(
        f, m, (((1,), (1,)), ((), ())),
        preferred_element_type=jnp.float32)
    t = m2 - 2.0 * prod
    acc = t[:, :_W]
    for k in range(1, _KB // _W):
        acc = jnp.minimum(acc, t[:, k * _W:(k + 1) * _W])
    rmin = jnp.min(acc, axis=1, keepdims=True)
    prev = jnp.where(j == 0, _BIG, o_ref[...])
    accmin = jnp.minimum(prev, rmin)
    f2 = jnp.sum(f * f, axis=1, keepdims=True)
    o_ref[...] = jnp.where(j == (K // _KB) - 1,
                           jnp.sqrt(jnp.maximum(accmin + f2, 0.0)), accmin)


def _stage2_body(pa_ref, ca_ref, s_ref, fs_ref, mx_ref):
    b = pl.program_id(0)
    pa = pa_ref[...]
    ca = ca_ref[...]
    dc = jax.lax.dot_general(
        pa, ca, (((1,), (1,)), ((), ())),
        preferred_element_type=jnp.float32)
    s = s_ref[...]
    nchunk = Q // _W
    t = jnp.full((_PB, 1), -_BIG, jnp.float32)
    for _ in range(KNN_K):
        acc = jnp.full((_PB, _W), _BIG, jnp.float32)
        for k in range(nchunk):
            c = dc[:, k * _W:(k + 1) * _W]
            acc = jnp.minimum(acc, jnp.where(c > t, c, _BIG))
        t = jnp.min(acc, axis=1, keepdims=True)
    z = jnp.zeros((_PB, _W), jnp.float32)
    cnt_lt, sum_lt, cnt_eq, sum_eq = z, z, z, z
    for k in range(nchunk):
        c = dc[:, k * _W:(k + 1) * _W]
        sb = s[:, k * _W:(k + 1) * _W]
        lt = c < t
        eq = c == t
        cnt_lt = cnt_lt + jnp.where(lt, 1.0, 0.0)
        sum_lt = sum_lt + jnp.where(lt, sb, 0.0)
        cnt_eq = cnt_eq + jnp.where(eq, 1.0, 0.0)
        sum_eq = sum_eq + jnp.where(eq, sb, 0.0)
    cnt_lt = jnp.sum(cnt_lt, axis=1, keepdims=True)
    sum_lt = jnp.sum(sum_lt, axis=1, keepdims=True)
    cnt_eq = jnp.sum(cnt_eq, axis=1, keepdims=True)
    sum_eq = jnp.sum(sum_eq, axis=1, keepdims=True)
    full = (sum_lt + (KNN_K - cnt_lt) * sum_eq / cnt_eq) * (1.0 / KNN_K)
    fs_ref[...] = full
    blockmax = jnp.max(full, axis=0, keepdims=True)
    prevmx = jnp.where(b == 0, jnp.full((1, 1), -_BIG, jnp.float32),
                       mx_ref[...])
    mx_ref[...] = jnp.maximum(prevmx, blockmax)


@functools.partial(jax.jit)
def kernel(features, memory_features, centers, points):
    center_scores = pl.pallas_call(
        _stage1_body,
        grid=(Q // _QB, K // _KB),
        in_specs=[
            pl.BlockSpec((_QB, D), lambda i, j: (i, 0)),
            pl.BlockSpec((_KB, D), lambda i, j: (j, 0)),
        ],
        out_specs=pl.BlockSpec((_QB, 1), lambda i, j: (i, 0)),
        out_shape=jax.ShapeDtypeStruct((Q, 1), jnp.float32),
    )(features, memory_features)

    zeros_p = jnp.zeros((P, 4), jnp.float32)
    pa = jnp.concatenate(
        [-2.0 * points, jnp.ones((P, 1), jnp.float32), zeros_p], axis=1)
    c2 = jnp.sum(centers * centers, axis=1, keepdims=True)
    ca = jnp.concatenate(
        [centers, c2, jnp.zeros((Q, 4), jnp.float32)], axis=1)
    scores_row = center_scores.reshape(1, Q)

    full2d, mx = pl.pallas_call(
        _stage2_body,
        grid=(P // _PB,),
        in_specs=[
            pl.BlockSpec((_PB, 8), lambda b: (b, 0)),
            pl.BlockSpec((Q, 8), lambda b: (0, 0)),
            pl.BlockSpec((1, Q), lambda b: (0, 0)),
        ],
        out_specs=[
            pl.BlockSpec((_PB, 1), lambda b: (b, 0)),
            pl.BlockSpec((1, 1), lambda b: (0, 0)),
        ],
        out_shape=[
            jax.ShapeDtypeStruct((P, 1), jnp.float32),
            jax.ShapeDtypeStruct((1, 1), jnp.float32),
        ],
    )(pa, ca, scores_row)

    return full2d.reshape(P), mx.reshape(())


# speedup vs baseline: 32.2242x; 1.4155x over previous
import functools

import jax
import jax.numpy as jnp
from jax.experimental import pallas as pl

Q = 4096
K = 16384
D = 256
P = 16384
KNN_K = 10

_QB = 512
_KB = 2048
_PB = 256
_W = 128

_BIG = 3e38


def _stage1_body(f_ref, m_ref, o_ref):
    j = pl.program_id(1)
    f = f_ref[...]
    m = m_ref[...]
    ones = jnp.ones((1, D), jnp.float32)
    m2 = jax.lax.dot_general(
        ones, m * m, (((1,), (1,)), ((), ())),
        preferred_element_type=jnp.float32)
    prod = jax.lax.dot_general(
        f, m, (((1,), (1,)), ((), ())),
        preferred_element_type=jnp.float32)
    t = m2 - 2.0 * prod
    acc = t[:, :_W]
    for k in range(1, _KB // _W):
        acc = jnp.minimum(acc, t[:, k * _W:(k + 1) * _W])
    rmin = jnp.min(acc, axis=1, keepdims=True)
    prev = jnp.where(j == 0, _BIG, o_ref[...])
    accmin = jnp.minimum(prev, rmin)
    f2 = jnp.sum(f * f, axis=1, keepdims=True)
    o_ref[...] = jnp.where(j == (K // _KB) - 1,
                           jnp.sqrt(jnp.maximum(accmin + f2, 0.0)), accmin)


def _stage2_body(pa_ref, ca_ref, rhs_ref, fs_ref, mx_ref):
    b = pl.program_id(0)
    pa = pa_ref[...]
    ca = ca_ref[...]
    dc = jax.lax.dot_general(
        pa, ca, (((1,), (1,)), ((), ())),
        preferred_element_type=jnp.float32)
    nchunk = Q // _W
    big = jnp.full((_PB, _W), _BIG, jnp.float32)
    a0, a1, a2, a3 = big, big, big, big
    for k in range(nchunk):
        x = dc[:, k * _W:(k + 1) * _W]
        h0 = jnp.maximum(a0, x)
        a0 = jnp.minimum(a0, x)
        h1 = jnp.maximum(a1, h0)
        a1 = jnp.minimum(a1, h0)
        h2 = jnp.maximum(a2, h1)
        a2 = jnp.minimum(a2, h1)
        a3 = jnp.minimum(a3, h2)
    t = jnp.full((_PB, 1), -_BIG, jnp.float32)
    for _ in range(KNN_K):
        head = jnp.where(a0 > t, a0,
               jnp.where(a1 > t, a1,
               jnp.where(a2 > t, a2,
               jnp.where(a3 > t, a3, _BIG))))
        t = jnp.min(head, axis=1, keepdims=True)
    bad = jnp.any(a3 < t)

    def _direct(_):
        td = jnp.full((_PB, 1), -_BIG, jnp.float32)
        for _ in range(KNN_K):
            acc = jnp.full((_PB, _W), _BIG, jnp.float32)
            for k in range(nchunk):
                c = dc[:, k * _W:(k + 1) * _W]
                acc = jnp.minimum(acc, jnp.where(c > td, c, _BIG))
            td = jnp.min(acc, axis=1, keepdims=True)
        return td

    t10 = jax.lax.cond(bad, _direct, lambda _: t, None)
    rhs = rhs_ref[...]
    lt01 = jnp.where(dc < t10, 1.0, 0.0)
    eq01 = jnp.where(dc == t10, 1.0, 0.0)
    r_lt = jax.lax.dot_general(
        lt01, rhs, (((1,), (0,)), ((), ())),
        preferred_element_type=jnp.float32)
    r_eq = jax.lax.dot_general(
        eq01, rhs, (((1,), (0,)), ((), ())),
        preferred_element_type=jnp.float32)
    cnt_lt = r_lt[:, 0:1]
    sum_lt = r_lt[:, 1:2]
    cnt_eq = jnp.maximum(r_eq[:, 0:1], 1.0)
    sum_eq = r_eq[:, 1:2]
    full = (sum_lt + (KNN_K - cnt_lt) * sum_eq / cnt_eq) * (1.0 / KNN_K)
    fs_ref[...] = full
    blockmax = jnp.max(full, axis=0, keepdims=True)
    prevmx = jnp.where(b == 0, jnp.full((1, 1), -_BIG, jnp.float32),
                       mx_ref[...])
    mx_ref[...] = jnp.maximum(prevmx, blockmax)


@functools.partial(jax.jit)
def kernel(features, memory_features, centers, points):
    center_scores = pl.pallas_call(
        _stage1_body,
        grid=(Q // _QB, K // _KB),
        in_specs=[
            pl.BlockSpec((_QB, D), lambda i, j: (i, 0)),
            pl.BlockSpec((_KB, D), lambda i, j: (j, 0)),
        ],
        out_specs=pl.BlockSpec((_QB, 1), lambda i, j: (i, 0)),
        out_shape=jax.ShapeDtypeStruct((Q, 1), jnp.float32),
    )(features, memory_features)

    zeros_p = jnp.zeros((P, 4), jnp.float32)
    pa = jnp.concatenate(
        [-2.0 * points, jnp.ones((P, 1), jnp.float32), zeros_p], axis=1)
    c2 = jnp.sum(centers * centers, axis=1, keepdims=True)
    ca = jnp.concatenate(
        [centers, c2, jnp.zeros((Q, 4), jnp.float32)], axis=1)
    rhs = jnp.concatenate(
        [jnp.ones((Q, 1), jnp.float32), center_scores], axis=1)

    full2d, mx = pl.pallas_call(
        _stage2_body,
        grid=(P // _PB,),
        in_specs=[
            pl.BlockSpec((_PB, 8), lambda b: (b, 0)),
            pl.BlockSpec((Q, 8), lambda b: (0, 0)),
            pl.BlockSpec((Q, 2), lambda b: (0, 0)),
        ],
        out_specs=[
            pl.BlockSpec((_PB, 1), lambda b: (b, 0)),
            pl.BlockSpec((1, 1), lambda b: (0, 0)),
        ],
        out_shape=[
            jax.ShapeDtypeStruct((P, 1), jnp.float32),
            jax.ShapeDtypeStruct((1, 1), jnp.float32),
        ],
    )(pa, ca, rhs)

    return full2d.reshape(P), mx.reshape(())
